# baseline (speedup 1.0000x reference)
import functools
import jax
import jax.numpy as jnp
from jax import lax
from jax.experimental import pallas as pl
from jax.experimental.pallas import tpu as pltpu
from jax.experimental.pallas import tpu_sc as plsc

_ROWS = 16 * 2048
_C = 256
_N = _ROWS * _C
_NW = 32
_PER_W = _N // _NW
_CHUNK = 8192
_NCHUNK = _PER_W // _CHUNK


def _sc_kernel(pmu, pls_, fmu, fls, out, bpm, bpl, bfm, bfl, accv, sem):
    wid = lax.axis_index("s") * 2 + lax.axis_index("c")
    base = wid * _PER_W

    accv[...] = jnp.zeros((16,), jnp.float32)

    def chunk_body(ci, _):
        off = base + ci * _CHUNK
        pltpu.sync_copy(pmu.at[pl.ds(off, _CHUNK)], bpm)
        pltpu.sync_copy(pls_.at[pl.ds(off, _CHUNK)], bpl)
        pltpu.sync_copy(fmu.at[pl.ds(off, _CHUNK)], bfm)
        pltpu.sync_copy(fls.at[pl.ds(off, _CHUNK)], bfl)

        def vec_body(i, acc):
            s = i * 16
            p_ls = bpl[pl.ds(s, 16)]
            f_ls = bfl[pl.ds(s, 16)]
            d = bfm[pl.ds(s, 16)] - bpm[pl.ds(s, 16)]
            var_f = jnp.exp(2.0 * f_ls)
            inv_2vp = 0.5 * jnp.exp(-2.0 * p_ls)
            kl = (p_ls - f_ls - 0.5) + (var_f + d * d) * inv_2vp
            return acc + kl

        acc = lax.fori_loop(0, _CHUNK // 16, vec_body, accv[...])
        accv[...] = acc
        return 0

    lax.fori_loop(0, _NCHUNK, chunk_body, 0)
    pltpu.sync_copy(accv, out.at[wid])


def kernel(present_mu, present_log_sigma, future_mu, future_log_sigma):
    pmu = present_mu.reshape(_N)
    pls_ = present_log_sigma.reshape(_N)
    fmu = future_mu.reshape(_N)
    fls = future_log_sigma.reshape(_N)

    mesh = plsc.VectorSubcoreMesh(core_axis_name="c", subcore_axis_name="s")
    f = pl.kernel(
        _sc_kernel,
        mesh=mesh,
        out_type=jax.ShapeDtypeStruct((_NW, 16), jnp.float32),
        scratch_types=[
            pltpu.VMEM((_CHUNK,), jnp.float32),
            pltpu.VMEM((_CHUNK,), jnp.float32),
            pltpu.VMEM((_CHUNK,), jnp.float32),
            pltpu.VMEM((_CHUNK,), jnp.float32),
            pltpu.VMEM((16,), jnp.float32),
            pltpu.SemaphoreType.DMA,
        ],
    )
    partials = f(pmu, pls_, fmu, fls)
    return jnp.sum(partials) / jnp.float32(_ROWS)


# speedup vs baseline: 1.6215x; 1.6215x over previous
import jax
import jax.numpy as jnp
from jax import lax
from jax.experimental import pallas as pl
from jax.experimental.pallas import tpu as pltpu
from jax.experimental.pallas import tpu_sc as plsc

_ROWS = 16 * 2048
_C = 256
_N = _ROWS * _C
_NW = 32
_PER_W = _N // _NW
_CHUNK = 8192
_NCHUNK = _PER_W // _CHUNK
_UNROLL = 4


def _sc_kernel(pmu, pls_, fmu, fls, out,
               b0pm, b0pl, b0fm, b0fl,
               b1pm, b1pl, b1fm, b1fl,
               accv, sem0, sem1):
    wid = lax.axis_index("s") * 2 + lax.axis_index("c")
    base = wid * _PER_W
    srcs = (pmu, pls_, fmu, fls)
    bufs = ((b0pm, b0pl, b0fm, b0fl, sem0),
            (b1pm, b1pl, b1fm, b1fl, sem1))

    def issue(setid, ci):
        bs = bufs[setid]
        off = base + ci * _CHUNK
        for src, dst in zip(srcs, bs[:4]):
            pltpu.async_copy(src.at[pl.ds(off, _CHUNK)], dst, bs[4])

    def wait(setid, ci):
        bs = bufs[setid]
        off = base + ci * _CHUNK
        for src, dst in zip(srcs, bs[:4]):
            pltpu.make_async_copy(src.at[pl.ds(off, _CHUNK)], dst, bs[4]).wait()

    def compute(setid, acc):
        bpm, bpl, bfm, bfl, _ = bufs[setid]
        zero = jnp.zeros((16,), jnp.float32)

        def ubody(j, accs):
            news = []
            for u in range(_UNROLL):
                s = (j * _UNROLL + u) * 16
                p_ls = bpl[pl.ds(s, 16)]
                f_ls = bfl[pl.ds(s, 16)]
                d = bfm[pl.ds(s, 16)] - bpm[pl.ds(s, 16)]
                var_f = jnp.exp(2.0 * f_ls)
                inv_2vp = 0.5 * jnp.exp(-2.0 * p_ls)
                kl = (p_ls - f_ls - 0.5) + (var_f + d * d) * inv_2vp
                news.append(accs[u] + kl)
            return tuple(news)

        accs = lax.fori_loop(0, _CHUNK // (16 * _UNROLL), ubody,
                             (zero,) * _UNROLL)
        for a in accs:
            acc = acc + a
        return acc

    issue(0, 0)
    issue(1, 1)

    def pair_body(g, acc):
        ci0 = g * 2
        wait(0, ci0)
        acc = compute(0, acc)

        @pl.when(ci0 + 2 < _NCHUNK)
        def _():
            issue(0, ci0 + 2)

        wait(1, ci0 + 1)
        acc = compute(1, acc)

        @pl.when(ci0 + 3 < _NCHUNK)
        def _():
            issue(1, ci0 + 3)

        return acc

    acc = lax.fori_loop(0, _NCHUNK // 2, pair_body,
                        jnp.zeros((16,), jnp.float32))
    accv[...] = acc
    pltpu.sync_copy(accv, out.at[wid])


def kernel(present_mu, present_log_sigma, future_mu, future_log_sigma):
    pmu = present_mu.reshape(_N)
    pls_ = present_log_sigma.reshape(_N)
    fmu = future_mu.reshape(_N)
    fls = future_log_sigma.reshape(_N)

    mesh = plsc.VectorSubcoreMesh(core_axis_name="c", subcore_axis_name="s")
    f = pl.kernel(
        _sc_kernel,
        mesh=mesh,
        out_type=jax.ShapeDtypeStruct((_NW, 16), jnp.float32),
        scratch_types=(
            [pltpu.VMEM((_CHUNK,), jnp.float32) for _ in range(8)]
            + [pltpu.VMEM((16,), jnp.float32),
               pltpu.SemaphoreType.DMA,
               pltpu.SemaphoreType.DMA]
        ),
    )
    partials = f(pmu, pls_, fmu, fls)
    return jnp.sum(partials) / jnp.float32(_ROWS)


# speedup vs baseline: 1.7930x; 1.1058x over previous
import jax
import jax.numpy as jnp
from jax import lax
from jax.experimental import pallas as pl
from jax.experimental.pallas import tpu as pltpu
from jax.experimental.pallas import tpu_sc as plsc

_ROWS = 16 * 2048
_C = 256

_SC_ROWS = 6144
_TC_ROWS = _ROWS - _SC_ROWS

_NW = 32
_SC_N = _SC_ROWS * _C
_PER_W = _SC_N // _NW
_CHUNK = 8192
_NCHUNK = _PER_W // _CHUNK
_UNROLL = 4

_BLOCK_ROWS = 2048


def _sc_kernel(pmu, pls_, fmu, fls, out,
               b0pm, b0pl, b0fm, b0fl,
               b1pm, b1pl, b1fm, b1fl,
               accv, sem0, sem1):
    wid = lax.axis_index("s") * 2 + lax.axis_index("c")
    base = wid * _PER_W
    srcs = (pmu, pls_, fmu, fls)
    bufs = ((b0pm, b0pl, b0fm, b0fl, sem0),
            (b1pm, b1pl, b1fm, b1fl, sem1))

    def issue(setid, ci):
        bs = bufs[setid]
        off = base + ci * _CHUNK
        for src, dst in zip(srcs, bs[:4]):
            pltpu.async_copy(src.at[pl.ds(off, _CHUNK)], dst, bs[4])

    def wait(setid, ci):
        bs = bufs[setid]
        off = base + ci * _CHUNK
        for src, dst in zip(srcs, bs[:4]):
            pltpu.make_async_copy(src.at[pl.ds(off, _CHUNK)], dst, bs[4]).wait()

    def compute(setid, acc):
        bpm, bpl, bfm, bfl, _ = bufs[setid]
        zero = jnp.zeros((16,), jnp.float32)

        def ubody(j, accs):
            news = []
            for u in range(_UNROLL):
                s = (j * _UNROLL + u) * 16
                p_ls = bpl[pl.ds(s, 16)]
                f_ls = bfl[pl.ds(s, 16)]
                d = bfm[pl.ds(s, 16)] - bpm[pl.ds(s, 16)]
                var_f = jnp.exp(2.0 * f_ls)
                inv_2vp = 0.5 * jnp.exp(-2.0 * p_ls)
                kl = (p_ls - f_ls - 0.5) + (var_f + d * d) * inv_2vp
                news.append(accs[u] + kl)
            return tuple(news)

        accs = lax.fori_loop(0, _CHUNK // (16 * _UNROLL), ubody,
                             (zero,) * _UNROLL)
        for a in accs:
            acc = acc + a
        return acc

    issue(0, 0)
    issue(1, 1)

    def pair_body(g, acc):
        ci0 = g * 2
        wait(0, ci0)
        acc = compute(0, acc)

        @pl.when(ci0 + 2 < _NCHUNK)
        def _():
            issue(0, ci0 + 2)

        wait(1, ci0 + 1)
        acc = compute(1, acc)

        @pl.when(ci0 + 3 < _NCHUNK)
        def _():
            issue(1, ci0 + 3)

        return acc

    acc = lax.fori_loop(0, _NCHUNK // 2, pair_body,
                        jnp.zeros((16,), jnp.float32))
    accv[...] = acc
    pltpu.sync_copy(accv, out.at[wid])


def _tc_block_kernel(pmu_ref, pls_ref, fmu_ref, fls_ref, out_ref, acc_ref):
    i = pl.program_id(0)

    @pl.when(i == 0)
    def _init():
        acc_ref[...] = jnp.zeros_like(acc_ref)

    pls = pls_ref[...]
    fls = fls_ref[...]
    d = fmu_ref[...] - pmu_ref[...]
    var_f = jnp.exp(2.0 * fls)
    inv_2vp = 0.5 * jnp.exp(-2.0 * pls)
    kl = (pls - fls - 0.5) + (var_f + d * d) * inv_2vp
    acc_ref[...] += jnp.sum(kl.reshape(-1, 8, _C), axis=0)

    @pl.when(i == pl.num_programs(0) - 1)
    def _fin():
        out_ref[...] = jnp.sum(acc_ref[...])[None, None]


def kernel(present_mu, present_log_sigma, future_mu, future_log_sigma):
    pmu2 = present_mu.reshape(_ROWS, _C)
    pls2 = present_log_sigma.reshape(_ROWS, _C)
    fmu2 = future_mu.reshape(_ROWS, _C)
    fls2 = future_log_sigma.reshape(_ROWS, _C)

    pmu1 = present_mu.reshape(_ROWS * _C)
    pls1 = present_log_sigma.reshape(_ROWS * _C)
    fmu1 = future_mu.reshape(_ROWS * _C)
    fls1 = future_log_sigma.reshape(_ROWS * _C)

    mesh = plsc.VectorSubcoreMesh(core_axis_name="c", subcore_axis_name="s")
    sc_fn = pl.kernel(
        _sc_kernel,
        mesh=mesh,
        out_type=jax.ShapeDtypeStruct((_NW, 16), jnp.float32),
        scratch_types=(
            [pltpu.VMEM((_CHUNK,), jnp.float32) for _ in range(8)]
            + [pltpu.VMEM((16,), jnp.float32),
               pltpu.SemaphoreType.DMA,
               pltpu.SemaphoreType.DMA]
        ),
    )
    sc_partials = sc_fn(pmu1, pls1, fmu1, fls1)

    blk0 = _SC_ROWS // _BLOCK_ROWS
    in_spec = pl.BlockSpec((_BLOCK_ROWS, _C), lambda i: (i + blk0, 0))
    tc_out = pl.pallas_call(
        _tc_block_kernel,
        grid=(_TC_ROWS // _BLOCK_ROWS,),
        in_specs=[in_spec, in_spec, in_spec, in_spec],
        out_specs=pl.BlockSpec((1, 1), lambda i: (0, 0)),
        out_shape=jax.ShapeDtypeStruct((1, 1), jnp.float32),
        scratch_shapes=[pltpu.VMEM((8, _C), jnp.float32)],
    )(pmu2, pls2, fmu2, fls2)

    return (tc_out[0, 0] + jnp.sum(sc_partials)) / jnp.float32(_ROWS)


# speedup vs baseline: 5.0485x; 2.8156x over previous
import jax
import jax.numpy as jnp
from jax import lax
from jax.experimental import pallas as pl
from jax.experimental.pallas import tpu as pltpu
from jax.experimental.pallas import tpu_sc as plsc

_ROWS = 16 * 2048
_C = 256

_SC_ROWS = 6144
_TC_ROWS = _ROWS - _SC_ROWS

_NW = 32
_RPW = _SC_ROWS // _NW
_CH_ROWS = 32
_NCHUNK = _RPW // _CH_ROWS
_LANES = _C // 16

_BLOCK_ROWS = 2048


def _sc_kernel(pmu, pls_, fmu, fls, out,
               b0pm, b0pl, b0fm, b0fl,
               b1pm, b1pl, b1fm, b1fl,
               accv, sem0, sem1):
    wid = lax.axis_index("s") * 2 + lax.axis_index("c")
    base = wid * _RPW
    srcs = (pmu, pls_, fmu, fls)
    bufs = ((b0pm, b0pl, b0fm, b0fl, sem0),
            (b1pm, b1pl, b1fm, b1fl, sem1))

    def issue(setid, ci):
        bs = bufs[setid]
        r0 = base + ci * _CH_ROWS
        for src, dst in zip(srcs, bs[:4]):
            pltpu.async_copy(src.at[pl.ds(r0, _CH_ROWS), :], dst, bs[4])

    def wait(setid, ci):
        bs = bufs[setid]
        r0 = base + ci * _CH_ROWS
        for src, dst in zip(srcs, bs[:4]):
            pltpu.make_async_copy(
                src.at[pl.ds(r0, _CH_ROWS), :], dst, bs[4]).wait()

    def compute(setid, acc):
        bpm, bpl, bfm, bfl, _ = bufs[setid]
        zero = jnp.zeros((16,), jnp.float32)

        def row_body(r, accs):
            news = list(accs)
            for c in range(_LANES):
                sl = pl.ds(c * 16, 16)
                p_ls = bpl[r, sl]
                f_ls = bfl[r, sl]
                d = bfm[r, sl] - bpm[r, sl]
                var_f = jnp.exp(2.0 * f_ls)
                inv_2vp = 0.5 * jnp.exp(-2.0 * p_ls)
                kl = (p_ls - f_ls - 0.5) + (var_f + d * d) * inv_2vp
                news[c % 4] = news[c % 4] + kl
            return tuple(news)

        accs = lax.fori_loop(0, _CH_ROWS, row_body, (zero,) * 4)
        for a in accs:
            acc = acc + a
        return acc

    issue(0, 0)
    issue(1, 1)

    def pair_body(g, acc):
        ci0 = g * 2
        wait(0, ci0)
        acc = compute(0, acc)

        @pl.when(ci0 + 2 < _NCHUNK)
        def _():
            issue(0, ci0 + 2)

        wait(1, ci0 + 1)
        acc = compute(1, acc)

        @pl.when(ci0 + 3 < _NCHUNK)
        def _():
            issue(1, ci0 + 3)

        return acc

    acc = lax.fori_loop(0, _NCHUNK // 2, pair_body,
                        jnp.zeros((16,), jnp.float32))
    accv[...] = acc
    pltpu.sync_copy(accv, out.at[wid])


def _tc_block_kernel(pmu_ref, pls_ref, fmu_ref, fls_ref, out_ref, acc_ref):
    i = pl.program_id(0)

    @pl.when(i == 0)
    def _init():
        acc_ref[...] = jnp.zeros_like(acc_ref)

    pls = pls_ref[...]
    fls = fls_ref[...]
    d = fmu_ref[...] - pmu_ref[...]
    var_f = jnp.exp(2.0 * fls)
    inv_2vp = 0.5 * jnp.exp(-2.0 * pls)
    kl = (pls - fls - 0.5) + (var_f + d * d) * inv_2vp
    acc_ref[...] += jnp.sum(kl.reshape(-1, 8, _C), axis=0)

    @pl.when(i == pl.num_programs(0) - 1)
    def _fin():
        out_ref[...] = jnp.sum(acc_ref[...])[None, None]


def kernel(present_mu, present_log_sigma, future_mu, future_log_sigma):
    pmu2 = present_mu.reshape(_ROWS, _C)
    pls2 = present_log_sigma.reshape(_ROWS, _C)
    fmu2 = future_mu.reshape(_ROWS, _C)
    fls2 = future_log_sigma.reshape(_ROWS, _C)

    mesh = plsc.VectorSubcoreMesh(core_axis_name="c", subcore_axis_name="s")
    sc_fn = pl.kernel(
        _sc_kernel,
        mesh=mesh,
        out_type=jax.ShapeDtypeStruct((_NW, 16), jnp.float32),
        scratch_types=(
            [pltpu.VMEM((_CH_ROWS, _C), jnp.float32) for _ in range(8)]
            + [pltpu.VMEM((16,), jnp.float32),
               pltpu.SemaphoreType.DMA,
               pltpu.SemaphoreType.DMA]
        ),
    )
    sc_partials = sc_fn(pmu2, pls2, fmu2, fls2)

    blk0 = _SC_ROWS // _BLOCK_ROWS
    in_spec = pl.BlockSpec((_BLOCK_ROWS, _C), lambda i: (i + blk0, 0))
    tc_out = pl.pallas_call(
        _tc_block_kernel,
        grid=(_TC_ROWS // _BLOCK_ROWS,),
        in_specs=[in_spec, in_spec, in_spec, in_spec],
        out_specs=pl.BlockSpec((1, 1), lambda i: (0, 0)),
        out_shape=jax.ShapeDtypeStruct((1, 1), jnp.float32),
        scratch_shapes=[pltpu.VMEM((8, _C), jnp.float32)],
    )(pmu2, pls2, fmu2, fls2)

    return (tc_out[0, 0] + jnp.sum(sc_partials)) / jnp.float32(_ROWS)


# speedup vs baseline: 7.1537x; 1.4170x over previous
import jax
import jax.numpy as jnp
from jax import lax
from jax.experimental import pallas as pl
from jax.experimental.pallas import tpu as pltpu

_ROWS = 16 * 2048
_C = 256
_CH_ROWS = 1024
_NCHUNK = _ROWS // _CH_ROWS
_NBUF = 4


def _kl_kernel(pmu, pls_, fmu, fls, out_ref, bufs, sems):
    srcs = (pmu, pls_, fmu, fls)

    def dma(b, ci, i):
        r0 = ci * _CH_ROWS
        return pltpu.make_async_copy(
            srcs[i].at[pl.ds(r0, _CH_ROWS), :], bufs.at[b, i], sems.at[b, i])

    def issue(b, ci):
        for i in range(4):
            dma(b, ci, i).start()

    def wait(b, ci):
        for i in range(4):
            dma(b, ci, i).wait()

    for b in range(_NBUF):
        issue(b, b)

    def group_body(g, acc):
        for b in range(_NBUF):
            ci = g * _NBUF + b
            wait(b, ci)
            pls = bufs[b, 1]
            fls = bufs[b, 3]
            d = bufs[b, 2] - bufs[b, 0]
            var_f = jnp.exp(2.0 * fls)
            inv_2vp = 0.5 * jnp.exp(-2.0 * pls)
            kl = (pls - fls - 0.5) + (var_f + d * d) * inv_2vp
            acc = acc + jnp.sum(kl.reshape(-1, 8, _C), axis=0)

            @pl.when(ci + _NBUF < _NCHUNK)
            def _():
                issue(b, ci + _NBUF)
        return acc

    acc = lax.fori_loop(0, _NCHUNK // _NBUF, group_body,
                        jnp.zeros((8, _C), jnp.float32))
    out_ref[...] = jnp.sum(acc)[None, None]


def kernel(present_mu, present_log_sigma, future_mu, future_log_sigma):
    pmu = present_mu.reshape(_ROWS, _C)
    pls_ = present_log_sigma.reshape(_ROWS, _C)
    fmu = future_mu.reshape(_ROWS, _C)
    fls = future_log_sigma.reshape(_ROWS, _C)

    hbm_spec = pl.BlockSpec(memory_space=pl.ANY)
    out = pl.pallas_call(
        _kl_kernel,
        in_specs=[hbm_spec, hbm_spec, hbm_spec, hbm_spec],
        out_specs=pl.BlockSpec(memory_space=pltpu.MemorySpace.VMEM),
        out_shape=jax.ShapeDtypeStruct((1, 1), jnp.float32),
        scratch_shapes=[
            pltpu.VMEM((_NBUF, 4, _CH_ROWS, _C), jnp.float32),
            pltpu.SemaphoreType.DMA((_NBUF, 4)),
        ],
    )(pmu, pls_, fmu, fls)
    return out[0, 0] / jnp.float32(_ROWS)
